# baseline (speedup 1.0000x reference)
import functools

import jax
import jax.numpy as jnp
from jax import lax
from jax.experimental import pallas as pl
from jax.experimental.pallas import tpu as pltpu
from jax.experimental.pallas import tpu_sc as plsc

VOCAB = 100000
EMB = 64
B = 1024
MAX_NORM = 1.0


def _make_sc_gather():
    info = plsc.get_sparse_core_info()
    nc, ns = info.num_cores, info.num_subcores
    nw = nc * ns
    b_per_w = B // nw
    mesh = plsc.VectorSubcoreMesh(core_axis_name="c", subcore_axis_name="s")

    @functools.partial(
        pl.kernel,
        mesh=mesh,
        out_type=jax.ShapeDtypeStruct((B, EMB), jnp.float32),
        scratch_types=[
            pltpu.VMEM((b_per_w,), jnp.int32),
            pltpu.VMEM((b_per_w, EMB), jnp.float32),
            pltpu.SemaphoreType.DMA,
        ],
        compiler_params=pltpu.CompilerParams(use_tc_tiling_on_sc=False),
    )
    def gather_k(table_hbm, idx_hbm, out_hbm, idx_v, rows_v, sem):
        wid = lax.axis_index("s") * nc + lax.axis_index("c")
        base = wid * b_per_w
        pltpu.sync_copy(idx_hbm.at[pl.ds(base, b_per_w)], idx_v)
        pltpu.async_copy(table_hbm.at[idx_v], rows_v, sem).wait()
        pltpu.sync_copy(rows_v, out_hbm.at[pl.ds(base, b_per_w)])

    return gather_k


_sc_gather = _make_sc_gather()


NV_BLK = 2048


def _proj_body(emb_ref, fcwt_ref, fcb_ref, out_ref):
    emb = emb_ref[...]
    s = jnp.sum(emb * emb, axis=1, keepdims=True)
    n = jnp.sqrt(s)
    scale = jnp.where(n > MAX_NORM, MAX_NORM / (n + 1e-7), 1.0)
    wv = emb * scale
    rhs = jnp.concatenate([wv, jnp.ones((B, 1), jnp.float32)], axis=1)
    lhs = jnp.concatenate([fcwt_ref[...], fcb_ref[...]], axis=0)
    out_ref[...] = lax.dot_general(
        lhs, rhs, (((0,), (1,)), ((), ())),
        preferred_element_type=jnp.float32,
    )


def _make_proj():
    grid = (pl.cdiv(VOCAB, NV_BLK),)
    return pl.pallas_call(
        _proj_body,
        grid=grid,
        in_specs=[
            pl.BlockSpec((B, EMB), lambda j: (0, 0)),
            pl.BlockSpec((EMB, NV_BLK), lambda j: (0, j)),
            pl.BlockSpec((1, NV_BLK), lambda j: (0, j)),
        ],
        out_specs=pl.BlockSpec((NV_BLK, B), lambda j: (j, 0)),
        out_shape=jax.ShapeDtypeStruct((VOCAB, B), jnp.float32),
        compiler_params=pltpu.CompilerParams(
            dimension_semantics=("arbitrary",),
        ),
    )


_proj = _make_proj()


def kernel(word_vector, table, fc_w, fc_b):
    idx = word_vector[:, 0]
    gathered = _sc_gather(table, idx)
    out_t = _proj(gathered, fc_w.T, fc_b.reshape(1, VOCAB))
    return out_t.T
